# baseline (speedup 1.0000x reference)
import functools

import jax
import jax.numpy as jnp
from jax import lax
from jax.experimental import pallas as pl
from jax.experimental.pallas import tpu as pltpu
from jax.experimental.pallas import tpu_sc as plsc

N = 878
NP = 896
H = 8
HF = 256
FC = 256
E = 28096
D = H * HF

_ROWS_PER_TILE = NP // 16
_TILE_WORDS = _ROWS_PER_TILE * NP
_EHALF = E // 2
_LANES = 16


def _adj_body(ei_hbm, zero_hbm, a_hbm, src_v, dst_v, acc_v, sem0, sem1, sem2):
    c = lax.axis_index("c")
    s = lax.axis_index("s")
    lo = s * _ROWS_PER_TILE

    c0 = pltpu.async_copy(ei_hbm.at[0], src_v, sem0)
    c1 = pltpu.async_copy(ei_hbm.at[1], dst_v, sem1)
    c2 = pltpu.async_copy(zero_hbm, acc_v, sem2)
    c0.wait()
    c1.wait()
    c2.wait()

    ones = jnp.ones((_LANES,), jnp.float32)
    nrows = jnp.uint32(_ROWS_PER_TILE)
    nchunks = _EHALF // _LANES

    @plsc.parallel_loop(0, nchunks, unroll=8)
    def _scatter(i):
        off = (c * nchunks + i) * _LANES
        d16 = dst_v[pl.ds(off, _LANES)]
        s16 = src_v[pl.ds(off, _LANES)]
        rel = d16 - lo
        msk = lax.convert_element_type(rel, jnp.uint32) < nrows
        plsc.addupdate_scatter(acc_v, [rel * NP + s16], ones, mask=msk)

    pltpu.sync_copy(acc_v, a_hbm.at[c, pl.ds(s * _TILE_WORDS, _TILE_WORDS)])


@functools.cache
def _build_adj_fn():
    return functools.partial(
        pl.kernel,
        out_type=jax.ShapeDtypeStruct((2, NP * NP), jnp.float32),
        mesh=plsc.VectorSubcoreMesh(core_axis_name="c", subcore_axis_name="s"),
        compiler_params=pltpu.CompilerParams(needs_layout_passes=False),
        scratch_types=[
            pltpu.VMEM((E,), jnp.int32),
            pltpu.VMEM((E,), jnp.int32),
            pltpu.VMEM((_TILE_WORDS,), jnp.float32),
            pltpu.SemaphoreType.DMA,
            pltpu.SemaphoreType.DMA,
            pltpu.SemaphoreType.DMA,
        ],
    )(_adj_body)


def _fc_body(if_ref, wm_ref, wd_ref, bm_ref, bd_ref, x_ref):
    b = if_ref[...]
    xm = jnp.dot(b, wm_ref[...], preferred_element_type=jnp.float32) + bm_ref[...]
    xd = jnp.dot(b, wd_ref[...], preferred_element_type=jnp.float32) + bd_ref[...]
    row = lax.broadcasted_iota(jnp.int32, (N, 1), 0)
    x = jnp.where(row < 495, xm, xd)
    x_ref[...] = jnp.concatenate([x, jnp.zeros((NP - N, FC), jnp.float32)], axis=0)


def _fc(in_feat, wm, wd_p, bm, bd):
    return pl.pallas_call(
        _fc_body,
        out_shape=jax.ShapeDtypeStruct((NP, FC), jnp.float32),
    )(in_feat, wm, wd_p, bm, bd)


def _layer_body(nk, out_rows, out_dtype,
                x_ref, w_ref, alp_ref, arp_ref, a_ref, b_ref, out_ref, feat_scr):
    i = pl.program_id(0)
    part = jnp.dot(x_ref[...].astype(jnp.bfloat16),
                   w_ref[...].astype(jnp.bfloat16),
                   preferred_element_type=jnp.float32)

    @pl.when(i == 0)
    def _():
        feat_scr[...] = part

    @pl.when(i > 0)
    def _():
        feat_scr[...] = feat_scr[...] + part

    @pl.when(i == nk - 1)
    def _():
        fb = feat_scr[...].astype(jnp.bfloat16)
        dn = (((1,), (1,)), ((), ()))
        elt = lax.dot_general(alp_ref[...], fb, dn,
                              preferred_element_type=jnp.float32)
        ert = lax.dot_general(arp_ref[...], fb, dn,
                              preferred_element_type=jnp.float32)
        er = ert.T
        pl_ = jnp.exp(elt)
        ql_ = jnp.exp(0.2 * elt)
        pr_ = jnp.exp(er)
        qr_ = jnp.exp(0.2 * er)
        a = a_ref[...]
        one = jnp.bfloat16(1.0)
        for h in range(H):
            t1 = (pl_[h : h + 1, :].astype(jnp.bfloat16)
                  * pr_[:, h : h + 1].astype(jnp.bfloat16))
            t2 = (ql_[h : h + 1, :].astype(jnp.bfloat16)
                  * qr_[:, h : h + 1].astype(jnp.bfloat16))
            t = a * jnp.where(t1 > one, t1, t2)
            den = jnp.sum(t.astype(jnp.float32), axis=1, keepdims=True)
            acc = jnp.dot(t, fb[:, h * HF : (h + 1) * HF],
                          preferred_element_type=jnp.float32)
            o = acc * (1.0 / (den + 1e-9)) + b_ref[:, h * HF : (h + 1) * HF]
            out_ref[:, h * HF : (h + 1) * HF] = (
                jnp.maximum(o, 0.0)[:out_rows, :].astype(out_dtype))


def _layer(x, w, alp, arp, a, b, kb, out_rows=NP, out_dtype=jnp.float32):
    k = w.shape[0]
    nk = k // kb
    return pl.pallas_call(
        functools.partial(_layer_body, nk, out_rows, out_dtype),
        grid=(nk,),
        in_specs=[
            pl.BlockSpec((NP, kb), lambda i: (0, i)),
            pl.BlockSpec((kb, D), lambda i: (i, 0)),
            pl.BlockSpec((H, D), lambda i: (0, 0)),
            pl.BlockSpec((H, D), lambda i: (0, 0)),
            pl.BlockSpec((NP, NP), lambda i: (0, 0)),
            pl.BlockSpec((1, D), lambda i: (0, 0)),
        ],
        out_specs=pl.BlockSpec((out_rows, D), lambda i: (0, 0)),
        out_shape=jax.ShapeDtypeStruct((out_rows, D), out_dtype),
        scratch_shapes=[pltpu.VMEM((NP, D), jnp.float32)],
    )(x, w, alp, arp, a, b)


def _row_blockdiag(al):
    cols = lax.broadcasted_iota(jnp.int32, (H, D), 1)
    heads = lax.broadcasted_iota(jnp.int32, (H, D), 0)
    tiled = jnp.tile(al, (1, H))
    return jnp.where(cols // HF == heads, tiled, 0.0).astype(jnp.bfloat16)


def kernel(in_feat, edge_index, W_m, b_m, W_d, b_d, W1, al1, ar1, b1, W2, al2, ar2, b2):
    zeros_tile = jnp.zeros((_TILE_WORDS,), jnp.float32)
    parts = _build_adj_fn()(edge_index, zeros_tile)
    adj = (parts[0] + parts[1]).reshape(NP, NP).astype(jnp.bfloat16)

    wd_p = jnp.pad(W_d, ((0, 495 - 383), (0, 0)))
    x = _fc(in_feat, W_m, wd_p, b_m.reshape(1, FC), b_d.reshape(1, FC))

    h1 = _layer(x, W1, _row_blockdiag(al1), _row_blockdiag(ar1), adj,
                b1.reshape(1, D), kb=FC, out_dtype=jnp.bfloat16)
    return _layer(h1, W2, _row_blockdiag(al2), _row_blockdiag(ar2), adj,
                  b2.reshape(1, D), kb=FC, out_rows=N)


# speedup vs baseline: 1.4108x; 1.4108x over previous
import functools

import jax
import jax.numpy as jnp
from jax import lax
from jax.experimental import pallas as pl
from jax.experimental.pallas import tpu as pltpu
from jax.experimental.pallas import tpu_sc as plsc

N = 878
NP = 896
H = 8
HF = 256
FC = 256
E = 28096
D = H * HF

_ROWS_PER_TILE = NP // 32
_TILE_WORDS = _ROWS_PER_TILE * NP
_LANES = 16


def _adj_body(ei_hbm, zero_hbm, a_hbm, src_v, dst_v, acc_v, sem0, sem1, sem2):
    wid = lax.axis_index("s") * 2 + lax.axis_index("c")
    lo = wid * _ROWS_PER_TILE

    c0 = pltpu.async_copy(ei_hbm.at[0], src_v, sem0)
    c1 = pltpu.async_copy(ei_hbm.at[1], dst_v, sem1)
    c2 = pltpu.async_copy(zero_hbm, acc_v, sem2)
    c0.wait()
    c1.wait()
    c2.wait()

    ones = jnp.ones((_LANES,), jnp.float32)
    nrows = jnp.uint32(_ROWS_PER_TILE)

    @plsc.parallel_loop(0, E // _LANES, unroll=8)
    def _scatter(i):
        off = i * _LANES
        d16 = dst_v[pl.ds(off, _LANES)]
        s16 = src_v[pl.ds(off, _LANES)]
        rel = d16 - lo
        msk = lax.convert_element_type(rel, jnp.uint32) < nrows
        plsc.addupdate_scatter(acc_v, [rel * NP + s16], ones, mask=msk)

    pltpu.sync_copy(acc_v, a_hbm.at[pl.ds(wid * _TILE_WORDS, _TILE_WORDS)])


@functools.cache
def _build_adj_fn():
    return functools.partial(
        pl.kernel,
        out_type=jax.ShapeDtypeStruct((NP * NP,), jnp.float32),
        mesh=plsc.VectorSubcoreMesh(core_axis_name="c", subcore_axis_name="s"),
        compiler_params=pltpu.CompilerParams(needs_layout_passes=False),
        scratch_types=[
            pltpu.VMEM((E,), jnp.int32),
            pltpu.VMEM((E,), jnp.int32),
            pltpu.VMEM((_TILE_WORDS,), jnp.float32),
            pltpu.SemaphoreType.DMA,
            pltpu.SemaphoreType.DMA,
            pltpu.SemaphoreType.DMA,
        ],
    )(_adj_body)


def _fc_body(if_ref, wm_ref, wd_ref, bm_ref, bd_ref, x_ref):
    b = if_ref[...]
    xm = jnp.dot(b, wm_ref[...], preferred_element_type=jnp.float32) + bm_ref[...]
    xd = jnp.dot(b, wd_ref[...], preferred_element_type=jnp.float32) + bd_ref[...]
    row = lax.broadcasted_iota(jnp.int32, (N, 1), 0)
    x = jnp.where(row < 495, xm, xd)
    x_ref[...] = jnp.concatenate([x, jnp.zeros((NP - N, FC), jnp.float32)], axis=0)


def _fc(in_feat, wm, wd_p, bm, bd):
    return pl.pallas_call(
        _fc_body,
        out_shape=jax.ShapeDtypeStruct((NP, FC), jnp.float32),
    )(in_feat, wm, wd_p, bm, bd)


def _layer_body(nk, out_rows, out_dtype,
                x_ref, w_ref, alp_ref, arp_ref, a_ref, b_ref, out_ref, feat_scr):
    i = pl.program_id(0)
    part = jnp.dot(x_ref[...].astype(jnp.bfloat16),
                   w_ref[...].astype(jnp.bfloat16),
                   preferred_element_type=jnp.float32)

    @pl.when(i == 0)
    def _():
        feat_scr[...] = part

    @pl.when(i > 0)
    def _():
        feat_scr[...] = feat_scr[...] + part

    @pl.when(i == nk - 1)
    def _():
        fb = feat_scr[...].astype(jnp.bfloat16)
        dn = (((1,), (1,)), ((), ()))
        elt = lax.dot_general(alp_ref[...], fb, dn,
                              preferred_element_type=jnp.float32)
        ert = lax.dot_general(arp_ref[...], fb, dn,
                              preferred_element_type=jnp.float32)
        er = ert.T
        pl_ = jnp.exp(elt)
        ql_ = jnp.exp(0.2 * elt)
        pr_ = jnp.exp(er)
        qr_ = jnp.exp(0.2 * er)
        a = a_ref[...].astype(jnp.bfloat16)
        one = jnp.bfloat16(1.0)
        onescol = jnp.ones((NP, 128), jnp.bfloat16)
        for h in range(H):
            t1 = (pl_[h : h + 1, :].astype(jnp.bfloat16)
                  * pr_[:, h : h + 1].astype(jnp.bfloat16))
            t2 = (ql_[h : h + 1, :].astype(jnp.bfloat16)
                  * qr_[:, h : h + 1].astype(jnp.bfloat16))
            t = a * jnp.where(t1 > one, t1, t2)
            den = jnp.dot(t, onescol,
                          preferred_element_type=jnp.float32)[:, 0:1]
            acc = jnp.dot(t, fb[:, h * HF : (h + 1) * HF],
                          preferred_element_type=jnp.float32)
            o = acc * (1.0 / (den + 1e-9)) + b_ref[:, h * HF : (h + 1) * HF]
            out_ref[:, h * HF : (h + 1) * HF] = (
                jnp.maximum(o, 0.0)[:out_rows, :].astype(out_dtype))


def _layer(x, w, alp, arp, a, b, kb, out_rows=NP, out_dtype=jnp.float32):
    k = w.shape[0]
    nk = k // kb
    return pl.pallas_call(
        functools.partial(_layer_body, nk, out_rows, out_dtype),
        grid=(nk,),
        in_specs=[
            pl.BlockSpec((NP, kb), lambda i: (0, i)),
            pl.BlockSpec((kb, D), lambda i: (i, 0)),
            pl.BlockSpec((H, D), lambda i: (0, 0)),
            pl.BlockSpec((H, D), lambda i: (0, 0)),
            pl.BlockSpec((NP, NP), lambda i: (0, 0)),
            pl.BlockSpec((1, D), lambda i: (0, 0)),
        ],
        out_specs=pl.BlockSpec((out_rows, D), lambda i: (0, 0)),
        out_shape=jax.ShapeDtypeStruct((out_rows, D), out_dtype),
        scratch_shapes=[pltpu.VMEM((NP, D), jnp.float32)],
    )(x, w, alp, arp, a, b)


def _row_blockdiag(al):
    cols = lax.broadcasted_iota(jnp.int32, (H, D), 1)
    heads = lax.broadcasted_iota(jnp.int32, (H, D), 0)
    tiled = jnp.tile(al, (1, H))
    return jnp.where(cols // HF == heads, tiled, 0.0).astype(jnp.bfloat16)


def kernel(in_feat, edge_index, W_m, b_m, W_d, b_d, W1, al1, ar1, b1, W2, al2, ar2, b2):
    zeros_tile = jnp.zeros((_TILE_WORDS,), jnp.float32)
    adj = _build_adj_fn()(edge_index, zeros_tile).reshape(NP, NP)

    wd_p = jnp.pad(W_d, ((0, 495 - 383), (0, 0)))
    x = _fc(in_feat, W_m, wd_p, b_m.reshape(1, FC), b_d.reshape(1, FC))

    h1 = _layer(x, W1, _row_blockdiag(al1), _row_blockdiag(ar1), adj,
                b1.reshape(1, D), kb=FC, out_dtype=jnp.bfloat16)
    return _layer(h1, W2, _row_blockdiag(al2), _row_blockdiag(ar2), adj,
                  b2.reshape(1, D), kb=512, out_rows=N)
